# baseline (device time: 51331 ns/iter reference)
import jax
import jax.numpy as jnp
from jax import lax
from jax.experimental import pallas as pl
from jax.experimental.pallas import tpu as pltpu

N_DEV = 4
B, SQ, SKV, HLOC, DH = 2, 256, 256, 4, 64
DMODEL = 512
HD = HLOC * DH


def kernel(x, Wq, K_ext, V_ext, Wo):
    def body(x_ref, wq_ref, k_ref, v_ref, wo_ref, out_ref,
             ctx_ref, comm_ref, send_sems, recv_sems):
        my = lax.axis_index("i")
        left = (my + N_DEV - 1) % N_DEV
        right = (my + 1) % N_DEV

        barrier_sem = pltpu.get_barrier_semaphore()
        for nbr in (left, right):
            pl.semaphore_signal(
                barrier_sem, inc=1,
                device_id=(nbr,), device_id_type=pl.DeviceIdType.MESH,
            )
        pl.semaphore_wait(barrier_sem, 2)

        wq_loc = wq_ref[:, pl.ds(my * HD, HD)]
        wo_loc = wo_ref[pl.ds(my * HD, HD), :]

        qi = lax.broadcasted_iota(jnp.int32, (SQ, SKV), 0)
        ki = lax.broadcasted_iota(jnp.int32, (SQ, SKV), 1)
        mask = (jnp.abs(qi - ki) <= 128) | (ki < 32) | (qi < 32)

        for b in range(B):
            qb = jnp.dot(x_ref[b], wq_loc, preferred_element_type=jnp.float32)
            kb = k_ref[b].reshape(SKV, HD)
            vb = v_ref[b].reshape(SKV, HD)
            for h in range(HLOC):
                qh = qb[:, h * DH:(h + 1) * DH]
                kh = kb[:, h * DH:(h + 1) * DH]
                s = lax.dot_general(
                    qh, kh, (((1,), (1,)), ((), ())),
                    preferred_element_type=jnp.float32,
                ) * 0.125
                s = jnp.where(mask, s, -1e9)
                m = jnp.max(s, axis=1, keepdims=True)
                w = jnp.exp(s - m)
                w = w / jnp.sum(w, axis=1, keepdims=True)
                ctx_ref[b, :, h * DH:(h + 1) * DH] = jnp.dot(
                    w, vb[:, h * DH:(h + 1) * DH],
                    preferred_element_type=jnp.float32,
                )

        for b in range(B):
            part = jnp.dot(ctx_ref[b], wo_loc,
                           preferred_element_type=jnp.float32)
            out_ref[b] = part
            comm_ref[0, b] = part

        for h in range(N_DEV - 1):
            rdma = pltpu.make_async_remote_copy(
                src_ref=comm_ref.at[h],
                dst_ref=comm_ref.at[h + 1],
                send_sem=send_sems.at[h],
                recv_sem=recv_sems.at[h],
                device_id=(right,),
                device_id_type=pl.DeviceIdType.MESH,
            )
            rdma.start()
            rdma.wait()
            out_ref[...] = out_ref[...] + comm_ref[h + 1]

    return pl.pallas_call(
        body,
        out_shape=jax.ShapeDtypeStruct((B, SQ, DMODEL), jnp.float32),
        in_specs=[pl.BlockSpec(memory_space=pltpu.VMEM)] * 5,
        out_specs=pl.BlockSpec(memory_space=pltpu.VMEM),
        scratch_shapes=[
            pltpu.VMEM((B, SQ, HD), jnp.float32),
            pltpu.VMEM((N_DEV, B, SQ, DMODEL), jnp.float32),
            pltpu.SemaphoreType.DMA((N_DEV - 1,)),
            pltpu.SemaphoreType.DMA((N_DEV - 1,)),
        ],
        compiler_params=pltpu.CompilerParams(collective_id=0),
    )(x, Wq, K_ext, V_ext, Wo)


# device time: 20322 ns/iter; 2.5259x vs baseline; 2.5259x over previous
import jax
import jax.numpy as jnp
from jax import lax
from jax.experimental import pallas as pl
from jax.experimental.pallas import tpu as pltpu

N_DEV = 4
B, SQ, SKV, HLOC, DH = 2, 256, 256, 4, 64
DMODEL = 512
HD = HLOC * DH


def kernel(x, Wq, K_ext, V_ext, Wo):
    def body(x_ref, wq_ref, k_ref, v_ref, wo_ref, out_ref,
             ctx_ref, ctxb_ref, wob_ref, comm_ref, send_sems, recv_sems):
        my = lax.axis_index("i")
        left = (my + N_DEV - 1) % N_DEV
        right = (my + 1) % N_DEV
        diag = (my + 2) % N_DEV

        barrier_sem = pltpu.get_barrier_semaphore()
        for nbr in (left, right, diag):
            pl.semaphore_signal(
                barrier_sem, inc=1,
                device_id=(nbr,), device_id_type=pl.DeviceIdType.MESH,
            )
        pl.semaphore_wait(barrier_sem, 3)

        wq_loc = wq_ref[:, pl.ds(my * HD, HD)]

        qi = lax.broadcasted_iota(jnp.int32, (SQ, SKV), 0)
        ki = lax.broadcasted_iota(jnp.int32, (SQ, SKV), 1)
        mask = (jnp.abs(qi - ki) <= 128) | (ki < 32) | (qi < 32)

        for b in range(B):
            qb = jnp.dot(x_ref[b], wq_loc, preferred_element_type=jnp.float32)
            kb = k_ref[b].reshape(SKV, HD)
            vb = v_ref[b].reshape(SKV, HD)
            for h in range(HLOC):
                qh = qb[:, h * DH:(h + 1) * DH]
                kh = kb[:, h * DH:(h + 1) * DH]
                s = lax.dot_general(
                    qh, kh, (((1,), (1,)), ((), ())),
                    preferred_element_type=jnp.float32,
                ) * 0.125
                s = jnp.where(mask, s, -1e9)
                m = jnp.max(s, axis=1, keepdims=True)
                w = jnp.exp(s - m)
                w = w / jnp.sum(w, axis=1, keepdims=True)
                ctx_ref[b, :, h * DH:(h + 1) * DH] = jnp.dot(
                    w, vb[:, h * DH:(h + 1) * DH],
                    preferred_element_type=jnp.float32,
                )

        ctxb_ref[...] = ctx_ref[...].astype(jnp.bfloat16)
        rdmas = {}
        for dest, slot in ((right, 0), (left, 1), (diag, 2)):
            r = pltpu.make_async_remote_copy(
                src_ref=ctxb_ref,
                dst_ref=comm_ref.at[slot],
                send_sem=send_sems.at[slot],
                recv_sem=recv_sems.at[slot],
                device_id=(dest,),
                device_id_type=pl.DeviceIdType.MESH,
            )
            r.start()
            rdmas[slot] = r

        wo_loc = wo_ref[pl.ds(my * HD, HD), :]
        for b in range(B):
            out_ref[b] = jnp.dot(ctx_ref[b], wo_loc,
                                 preferred_element_type=jnp.float32)
        wob_ref[...] = wo_ref[...].astype(jnp.bfloat16)

        for slot, src in ((0, left), (1, right), (2, diag)):
            rdmas[slot].wait_recv()
            wo_s = wob_ref[pl.ds(src * HD, HD), :]
            for b in range(B):
                out_ref[b] = out_ref[b] + jnp.dot(
                    comm_ref[slot, b], wo_s,
                    preferred_element_type=jnp.float32,
                )

        for slot in (0, 1, 2):
            rdmas[slot].wait_send()

    return pl.pallas_call(
        body,
        out_shape=jax.ShapeDtypeStruct((B, SQ, DMODEL), jnp.float32),
        in_specs=[pl.BlockSpec(memory_space=pltpu.VMEM)] * 5,
        out_specs=pl.BlockSpec(memory_space=pltpu.VMEM),
        scratch_shapes=[
            pltpu.VMEM((B, SQ, HD), jnp.float32),
            pltpu.VMEM((B, SQ, HD), jnp.bfloat16),
            pltpu.VMEM((N_DEV * HD, DMODEL), jnp.bfloat16),
            pltpu.VMEM((3, B, SQ, HD), jnp.bfloat16),
            pltpu.SemaphoreType.DMA((3,)),
            pltpu.SemaphoreType.DMA((3,)),
        ],
        compiler_params=pltpu.CompilerParams(collective_id=0),
    )(x, Wq, K_ext, V_ext, Wo)


# device time: 19799 ns/iter; 2.5926x vs baseline; 1.0264x over previous
import jax
import jax.numpy as jnp
from jax import lax
from jax.experimental import pallas as pl
from jax.experimental.pallas import tpu as pltpu

N_DEV = 4
B, SQ, SKV, HLOC, DH = 2, 256, 256, 4, 64
DMODEL = 512
HD = HLOC * DH


def kernel(x, Wq, K_ext, V_ext, Wo):
    def body(x_ref, wq_ref, k_ref, v_ref, wo_ref, out_ref,
             ctxb_ref, wob_ref, comm_ref, send_sems, recv_sems):
        my = lax.axis_index("i")
        left = (my + N_DEV - 1) % N_DEV
        right = (my + 1) % N_DEV
        diag = (my + 2) % N_DEV

        barrier_sem = pltpu.get_barrier_semaphore()
        for nbr in (left, right, diag):
            pl.semaphore_signal(
                barrier_sem, inc=1,
                device_id=(nbr,), device_id_type=pl.DeviceIdType.MESH,
            )
        pl.semaphore_wait(barrier_sem, 3)

        wq_loc = wq_ref[:, pl.ds(my * HD, HD)].astype(jnp.bfloat16)

        qi = lax.broadcasted_iota(jnp.int32, (SQ, SKV), 0)
        ki = lax.broadcasted_iota(jnp.int32, (SQ, SKV), 1)
        mask = (jnp.abs(qi - ki) <= 128) | (ki < 32) | (qi < 32)

        for b in range(B):
            xb = x_ref[b].astype(jnp.bfloat16)
            qb = jnp.dot(xb, wq_loc,
                         preferred_element_type=jnp.float32
                         ).astype(jnp.bfloat16)
            kb = k_ref[b].reshape(SKV, HD).astype(jnp.bfloat16)
            vb = v_ref[b].reshape(SKV, HD).astype(jnp.bfloat16)
            for h in range(HLOC):
                qh = qb[:, h * DH:(h + 1) * DH]
                kh = kb[:, h * DH:(h + 1) * DH]
                s = lax.dot_general(
                    qh, kh, (((1,), (1,)), ((), ())),
                    preferred_element_type=jnp.float32,
                ) * 0.125
                s = jnp.where(mask, s, -1e9)
                m = jnp.max(s, axis=1, keepdims=True)
                w = jnp.exp(s - m)
                w = (w / jnp.sum(w, axis=1, keepdims=True)).astype(jnp.bfloat16)
                ctxb_ref[b, :, h * DH:(h + 1) * DH] = jnp.dot(
                    w, vb[:, h * DH:(h + 1) * DH],
                    preferred_element_type=jnp.float32,
                ).astype(jnp.bfloat16)

        rdmas = {}
        for dest, slot in ((right, 0), (left, 1), (diag, 2)):
            r = pltpu.make_async_remote_copy(
                src_ref=ctxb_ref,
                dst_ref=comm_ref.at[slot],
                send_sem=send_sems.at[slot],
                recv_sem=recv_sems.at[slot],
                device_id=(dest,),
                device_id_type=pl.DeviceIdType.MESH,
            )
            r.start()
            rdmas[slot] = r

        wob_ref[...] = wo_ref[...].astype(jnp.bfloat16)
        wo_loc = wob_ref[pl.ds(my * HD, HD), :]
        for b in range(B):
            out_ref[b] = jnp.dot(ctxb_ref[b], wo_loc,
                                 preferred_element_type=jnp.float32)

        for slot, src in ((0, left), (1, right), (2, diag)):
            rdmas[slot].wait_recv()
            wo_s = wob_ref[pl.ds(src * HD, HD), :]
            for b in range(B):
                out_ref[b] = out_ref[b] + jnp.dot(
                    comm_ref[slot, b], wo_s,
                    preferred_element_type=jnp.float32,
                )

        for slot in (0, 1, 2):
            rdmas[slot].wait_send()

    return pl.pallas_call(
        body,
        out_shape=jax.ShapeDtypeStruct((B, SQ, DMODEL), jnp.float32),
        in_specs=[pl.BlockSpec(memory_space=pltpu.VMEM)] * 5,
        out_specs=pl.BlockSpec(memory_space=pltpu.VMEM),
        scratch_shapes=[
            pltpu.VMEM((B, SQ, HD), jnp.bfloat16),
            pltpu.VMEM((N_DEV * HD, DMODEL), jnp.bfloat16),
            pltpu.VMEM((3, B, SQ, HD), jnp.bfloat16),
            pltpu.SemaphoreType.DMA((3,)),
            pltpu.SemaphoreType.DMA((3,)),
        ],
        compiler_params=pltpu.CompilerParams(collective_id=0),
    )(x, Wq, K_ext, V_ext, Wo)


# device time: 15655 ns/iter; 3.2789x vs baseline; 1.2647x over previous
import jax
import jax.numpy as jnp
from jax import lax
from jax.experimental import pallas as pl
from jax.experimental.pallas import tpu as pltpu

N_DEV = 4
B, SQ, SKV, HLOC, DH = 2, 256, 256, 4, 64
DMODEL = 512
HD = HLOC * DH


def kernel(x, Wq, K_ext, V_ext, Wo):
    KT = jnp.transpose(K_ext, (0, 2, 3, 1))
    VT = jnp.transpose(V_ext, (0, 2, 3, 1))

    hbm = lambda a: pltpu.with_memory_space_constraint(a, pltpu.MemorySpace.HBM)
    x, Wq, KT, VT, Wo = hbm(x), hbm(Wq), hbm(KT), hbm(VT), hbm(Wo)

    def body(x_hbm, wq_hbm, kt_hbm, vt_hbm, wo_hbm, out_hbm,
             xv_ref, wqv_ref, ktv_ref, vtv_ref, wov_ref, wob_ref,
             ctxb_ref, acc_ref, comm_ref,
             in_sems, out_sem, send_sems, recv_sems):
        my = lax.axis_index("i")
        left = (my + N_DEV - 1) % N_DEV
        right = (my + 1) % N_DEV
        diag = (my + 2) % N_DEV

        barrier_sem = pltpu.get_barrier_semaphore()
        for nbr in (left, right, diag):
            pl.semaphore_signal(
                barrier_sem, inc=1,
                device_id=(nbr,), device_id_type=pl.DeviceIdType.MESH,
            )

        cp_x = pltpu.make_async_copy(x_hbm, xv_ref, in_sems.at[0])
        cp_wq = pltpu.make_async_copy(
            wq_hbm.at[:, pl.ds(my * HD, HD)], wqv_ref, in_sems.at[1])
        cp_kt = pltpu.make_async_copy(kt_hbm, ktv_ref, in_sems.at[2])
        cp_vt = pltpu.make_async_copy(vt_hbm, vtv_ref, in_sems.at[3])
        cp_wo = pltpu.make_async_copy(wo_hbm, wov_ref, in_sems.at[4])
        cp_x.start()
        cp_wq.start()
        cp_kt.start()
        cp_vt.start()
        cp_wo.start()

        qi = lax.broadcasted_iota(jnp.int32, (SQ, SKV), 0)
        ki = lax.broadcasted_iota(jnp.int32, (SQ, SKV), 1)
        mask = (jnp.abs(qi - ki) <= 128) | (ki < 32) | (qi < 32)

        cp_x.wait()
        cp_wq.wait()
        wq_loc = wqv_ref[...].astype(jnp.bfloat16)

        cp_kt.wait()
        cp_vt.wait()
        for b in range(B):
            xb = xv_ref[b].astype(jnp.bfloat16)
            qb = jnp.dot(xb, wq_loc,
                         preferred_element_type=jnp.float32
                         ).astype(jnp.bfloat16)
            for h in range(HLOC):
                qh = qb[:, h * DH:(h + 1) * DH]
                kth = ktv_ref[b, h].astype(jnp.bfloat16)
                s = jnp.dot(qh, kth,
                            preferred_element_type=jnp.float32) * 0.125
                s = jnp.where(mask, s, -1e9)
                m = jnp.max(s, axis=1, keepdims=True)
                w = jnp.exp(s - m)
                w = (w / jnp.sum(w, axis=1, keepdims=True)).astype(jnp.bfloat16)
                vth = vtv_ref[b, h].astype(jnp.bfloat16)
                ctxb_ref[b, :, h * DH:(h + 1) * DH] = lax.dot_general(
                    w, vth, (((1,), (1,)), ((), ())),
                    preferred_element_type=jnp.float32,
                ).astype(jnp.bfloat16)

        pl.semaphore_wait(barrier_sem, 3)

        rdmas = {}
        for dest, slot in ((diag, 2), (right, 0), (left, 1)):
            r = pltpu.make_async_remote_copy(
                src_ref=ctxb_ref,
                dst_ref=comm_ref.at[slot],
                send_sem=send_sems.at[slot],
                recv_sem=recv_sems.at[slot],
                device_id=(dest,),
                device_id_type=pl.DeviceIdType.MESH,
            )
            r.start()
            rdmas[slot] = r

        cp_wo.wait()
        wob_ref[...] = wov_ref[...].astype(jnp.bfloat16)
        wo_loc = wob_ref[pl.ds(my * HD, HD), :]
        for b in range(B):
            acc_ref[b] = jnp.dot(ctxb_ref[b], wo_loc,
                                 preferred_element_type=jnp.float32)

        for slot, src in ((0, left), (1, right), (2, diag)):
            rdmas[slot].wait_recv()
            wo_s = wob_ref[pl.ds(src * HD, HD), :]
            for b in range(B):
                acc_ref[b] = acc_ref[b] + jnp.dot(
                    comm_ref[slot, b], wo_s,
                    preferred_element_type=jnp.float32,
                )

        cp_out = pltpu.make_async_copy(acc_ref, out_hbm, out_sem)
        cp_out.start()
        for slot in (0, 1, 2):
            rdmas[slot].wait_send()
        cp_out.wait()

    return pl.pallas_call(
        body,
        out_shape=jax.ShapeDtypeStruct((B, SQ, DMODEL), jnp.float32),
        in_specs=[pl.BlockSpec(memory_space=pl.ANY)] * 5,
        out_specs=pl.BlockSpec(memory_space=pl.ANY),
        scratch_shapes=[
            pltpu.VMEM((B, SQ, DMODEL), jnp.float32),
            pltpu.VMEM((DMODEL, HD), jnp.float32),
            pltpu.VMEM((B, HLOC, DH, SKV), jnp.float32),
            pltpu.VMEM((B, HLOC, DH, SKV), jnp.float32),
            pltpu.VMEM((N_DEV * HD, DMODEL), jnp.float32),
            pltpu.VMEM((N_DEV * HD, DMODEL), jnp.bfloat16),
            pltpu.VMEM((B, SQ, HD), jnp.bfloat16),
            pltpu.VMEM((B, SQ, DMODEL), jnp.float32),
            pltpu.VMEM((3, B, SQ, HD), jnp.bfloat16),
            pltpu.SemaphoreType.DMA((5,)),
            pltpu.SemaphoreType.DMA,
            pltpu.SemaphoreType.DMA((3,)),
            pltpu.SemaphoreType.DMA((3,)),
        ],
        compiler_params=pltpu.CompilerParams(collective_id=0),
    )(x, Wq, KT, VT, Wo)


# device time: 13606 ns/iter; 3.7727x vs baseline; 1.1506x over previous
import jax
import jax.numpy as jnp
from jax import lax
from jax.experimental import pallas as pl
from jax.experimental.pallas import tpu as pltpu

N_DEV = 4
B, SQ, SKV, HLOC, DH = 2, 256, 256, 4, 64
DMODEL = 512
HD = HLOC * DH
NH = 2
HW = HD // NH


def kernel(x, Wq, K_ext, V_ext, Wo):
    KT = jnp.transpose(K_ext, (0, 2, 3, 1))
    VT = jnp.transpose(V_ext, (0, 2, 3, 1))

    hbm = lambda a: pltpu.with_memory_space_constraint(a, pltpu.MemorySpace.HBM)
    x, Wq, KT, VT, Wo = hbm(x), hbm(Wq), hbm(KT), hbm(VT), hbm(Wo)

    def body(x_hbm, wq_hbm, kt_hbm, vt_hbm, wo_hbm, out_hbm,
             xv_ref, wqv_ref, ktv_ref, vtv_ref, wov_ref, wob_ref,
             c00_ref, c01_ref, c10_ref, c11_ref, acc_ref, comm_ref,
             in_sems, out_sems, send_sems, recv_sems):
        my = lax.axis_index("i")
        left = (my + N_DEV - 1) % N_DEV
        right = (my + 1) % N_DEV
        diag = (my + 2) % N_DEV
        ctxb = ((c00_ref, c01_ref), (c10_ref, c11_ref))

        barrier_sem = pltpu.get_barrier_semaphore()
        for nbr in (left, right, diag):
            pl.semaphore_signal(
                barrier_sem, inc=1,
                device_id=(nbr,), device_id_type=pl.DeviceIdType.MESH,
            )

        cp_x0 = pltpu.make_async_copy(x_hbm.at[0], xv_ref.at[0], in_sems.at[0])
        cp_x1 = pltpu.make_async_copy(x_hbm.at[1], xv_ref.at[1], in_sems.at[5])
        cp_wq = pltpu.make_async_copy(
            wq_hbm.at[:, pl.ds(my * HD, HD)], wqv_ref, in_sems.at[1])
        cp_kt = pltpu.make_async_copy(kt_hbm, ktv_ref, in_sems.at[2])
        cp_vt = pltpu.make_async_copy(vt_hbm, vtv_ref, in_sems.at[3])
        cp_wo = pltpu.make_async_copy(wo_hbm, wov_ref, in_sems.at[4])
        cp_wq.start()
        cp_x0.start()
        cp_kt.start()
        cp_vt.start()
        cp_x1.start()
        cp_wo.start()

        qi = lax.broadcasted_iota(jnp.int32, (SQ, SKV), 0)
        ki = lax.broadcasted_iota(jnp.int32, (SQ, SKV), 1)
        mask = (jnp.abs(qi - ki) <= 128) | (ki < 32) | (qi < 32)

        cp_wq.wait()
        wq_loc = (wqv_ref[...] * 0.125).astype(jnp.bfloat16)
        cp_x0.wait()
        cp_kt.wait()
        cp_vt.wait()

        def attend_head(b, h):
            qh = qbs[b][:, h * DH:(h + 1) * DH]
            kth = ktv_ref[b, h].astype(jnp.bfloat16)
            s = jnp.dot(qh, kth, preferred_element_type=jnp.float32)
            w = jnp.exp(jnp.where(mask, s, -1e9))
            r = 1.0 / jnp.sum(w, axis=1, keepdims=True)
            vth = vtv_ref[b, h].astype(jnp.bfloat16)
            ctx = lax.dot_general(
                w.astype(jnp.bfloat16), vth, (((1,), (1,)), ((), ())),
                preferred_element_type=jnp.float32,
            )
            return ctx * r

        def send_wave(b, half):
            rs = {}
            for dest, slot in ((diag, 2), (right, 0), (left, 1)):
                r = pltpu.make_async_remote_copy(
                    src_ref=ctxb[b][half],
                    dst_ref=comm_ref.at[slot, b, half],
                    send_sem=send_sems.at[slot, b, half],
                    recv_sem=recv_sems.at[slot, b, half],
                    device_id=(dest,),
                    device_id_type=pl.DeviceIdType.MESH,
                )
                r.start()
                rs[slot] = r
            return rs

        qbs = [None, None]
        qbs[0] = jnp.dot(xv_ref[0].astype(jnp.bfloat16), wq_loc,
                         preferred_element_type=jnp.float32
                         ).astype(jnp.bfloat16)

        rd = {}
        for hh in range(NH):
            ctxb[0][0][:, hh * DH:(hh + 1) * DH] = (
                attend_head(0, hh).astype(jnp.bfloat16))
        pl.semaphore_wait(barrier_sem, 3)
        rd[(0, 0)] = send_wave(0, 0)

        for hh in range(NH):
            ctxb[0][1][:, hh * DH:(hh + 1) * DH] = (
                attend_head(0, NH + hh).astype(jnp.bfloat16))
        rd[(0, 1)] = send_wave(0, 1)

        cp_x1.wait()
        qbs[1] = jnp.dot(xv_ref[1].astype(jnp.bfloat16), wq_loc,
                         preferred_element_type=jnp.float32
                         ).astype(jnp.bfloat16)
        for half in range(2):
            for hh in range(NH):
                ctxb[1][half][:, hh * DH:(hh + 1) * DH] = (
                    attend_head(1, half * NH + hh).astype(jnp.bfloat16))
            rd[(1, half)] = send_wave(1, half)

        cp_wo.wait()
        wob_ref[...] = wov_ref[...].astype(jnp.bfloat16)
        for b in range(B):
            acc_ref[b] = sum(
                jnp.dot(ctxb[b][half][...],
                        wob_ref[pl.ds(my * HD + half * HW, HW), :],
                        preferred_element_type=jnp.float32)
                for half in range(2)
            )

        cp_outs = []
        for b in range(B):
            for half in range(2):
                for slot in (0, 1, 2):
                    rd[(b, half)][slot].wait_recv()
                acc_ref[b] = acc_ref[b] + sum(
                    jnp.dot(comm_ref[slot, b, half],
                            wob_ref[pl.ds(src * HD + half * HW, HW), :],
                            preferred_element_type=jnp.float32)
                    for slot, src in ((0, left), (1, right), (2, diag))
                )
            cp = pltpu.make_async_copy(
                acc_ref.at[b], out_hbm.at[b], out_sems.at[b])
            cp.start()
            cp_outs.append(cp)

        for key in rd:
            for slot in (0, 1, 2):
                rd[key][slot].wait_send()
        for cp in cp_outs:
            cp.wait()

    return pl.pallas_call(
        body,
        out_shape=jax.ShapeDtypeStruct((B, SQ, DMODEL), jnp.float32),
        in_specs=[pl.BlockSpec(memory_space=pl.ANY)] * 5,
        out_specs=pl.BlockSpec(memory_space=pl.ANY),
        scratch_shapes=[
            pltpu.VMEM((B, SQ, DMODEL), jnp.float32),
            pltpu.VMEM((DMODEL, HD), jnp.float32),
            pltpu.VMEM((B, HLOC, DH, SKV), jnp.float32),
            pltpu.VMEM((B, HLOC, DH, SKV), jnp.float32),
            pltpu.VMEM((N_DEV * HD, DMODEL), jnp.float32),
            pltpu.VMEM((N_DEV * HD, DMODEL), jnp.bfloat16),
            pltpu.VMEM((SQ, HW), jnp.bfloat16),
            pltpu.VMEM((SQ, HW), jnp.bfloat16),
            pltpu.VMEM((SQ, HW), jnp.bfloat16),
            pltpu.VMEM((SQ, HW), jnp.bfloat16),
            pltpu.VMEM((B, SQ, DMODEL), jnp.float32),
            pltpu.VMEM((3, B, 2, SQ, HW), jnp.bfloat16),
            pltpu.SemaphoreType.DMA((6,)),
            pltpu.SemaphoreType.DMA((B,)),
            pltpu.SemaphoreType.DMA((3, B, 2)),
            pltpu.SemaphoreType.DMA((3, B, 2)),
        ],
        compiler_params=pltpu.CompilerParams(collective_id=0),
    )(x, Wq, KT, VT, Wo)
